# baseline (speedup 1.0000x reference)
import jax
import jax.numpy as jnp
from jax import lax
from jax.experimental import pallas as pl
from jax.experimental.pallas import tpu as pltpu
from jax.experimental.pallas import tpu_sc as plsc

_B, _L, _D = 16, 4096, 512
_SPLIT = 256

_CH = 64
_NCH = _L // _CH
_DH = _D // 2
_NJ = _DH // 16
_U = 4

_BLK = 256
_NBUF = 6


def _sc_prefix_body(seq, begin_h, end_h, out, buf0, buf1, bev, env, outb,
                    sem0, sem1):
    c = lax.axis_index("c")
    s = lax.axis_index("s")
    wid = s * 2 + c
    row = wid % _B
    half = wid // _B

    pltpu.sync_copy(begin_h, bev.at[pl.ds(0, 16)])
    pltpu.sync_copy(begin_h, bev.at[pl.ds(16, 16)])
    pltpu.sync_copy(end_h, env.at[pl.ds(0, 16)])
    pltpu.sync_copy(end_h, env.at[pl.ds(16, 16)])
    w_end = jnp.minimum(env[pl.ds(row, 16)][0], _SPLIT)
    w_begin = jnp.minimum(bev[pl.ds(row, 16)][0], w_end)

    c0 = w_begin // _CH
    c1 = (w_end + _CH - 1) // _CH
    bufs = (buf0, buf1)
    sems = (sem0, sem1)

    def issue(ci, b):
        cc = jnp.clip(ci, 0, _NCH - 1)
        pltpu.async_copy(
            seq.at[row, pl.ds(cc * _CH, _CH), pl.ds(half * _DH, _DH)],
            bufs[b], sems[b])

    def wait(b):
        pltpu.make_async_copy(
            seq.at[0, pl.ds(0, _CH), pl.ds(0, _DH)], bufs[b], sems[b]).wait()

    issue(c0, 0)
    issue(c0 + 1, 1)

    def accum_chunk(ci, b, accs):
        buf = bufs[b]
        wait(b)
        lo = jnp.clip(w_begin - ci * _CH, 0, _CH)
        hi = jnp.clip(w_end - ci * _CH, 0, _CH)
        n = hi - lo
        n_main = n - n % _U

        def main_body(k, a):
            p = lo + k * _U
            for u in range(_U):
                a = tuple(a[j] + buf[p + u, pl.ds(j * 16, 16)]
                          for j in range(_NJ))
            return a

        accs = lax.fori_loop(0, n_main // _U, main_body, accs)

        def tail_body(p, a):
            return tuple(a[j] + buf[p, pl.ds(j * 16, 16)] for j in range(_NJ))

        accs = lax.fori_loop(lo + n_main, hi, tail_body, accs)
        issue(ci + 2, b)
        return accs

    def pair_body(k, accs):
        ci = c0 + 2 * k
        accs = accum_chunk(ci, 0, accs)
        accs = accum_chunk(ci + 1, 1, accs)
        return accs

    n_pairs = jnp.maximum(c1 - c0 + 1, 0) // 2
    accs0 = tuple(jnp.zeros((16,), jnp.float32) for _ in range(_NJ))
    accs = lax.fori_loop(0, n_pairs, pair_body, accs0)

    wait(0)
    wait(1)

    for j in range(_NJ):
        outb[pl.ds(j * 16, 16)] = accs[j]
    pltpu.sync_copy(outb, out.at[row, pl.ds(half * _DH, _DH)])


def _sc_prefix(seq, begin, end):
    mesh = plsc.VectorSubcoreMesh(
        core_axis_name="c", subcore_axis_name="s", num_cores=2,
        num_subcores=16)
    run = pl.kernel(
        _sc_prefix_body,
        out_type=jax.ShapeDtypeStruct((_B, _D), jnp.float32),
        mesh=mesh,
        scratch_types=[
            pltpu.VMEM((_CH, _DH), jnp.float32),
            pltpu.VMEM((_CH, _DH), jnp.float32),
            pltpu.VMEM((32,), jnp.int32),
            pltpu.VMEM((32,), jnp.int32),
            pltpu.VMEM((_DH,), jnp.float32),
            pltpu.SemaphoreType.DMA,
            pltpu.SemaphoreType.DMA,
        ],
    )
    return run(seq, begin, end)


def _tc_tail_body(begin_ref, end_ref, seq, sc_part, out, buf, sems):
    cnt = []
    blo = []
    for r in range(_B):
        sp = jnp.maximum(begin_ref[r], _SPLIT)
        lo_b = sp // _BLK
        hi_b = (jnp.maximum(end_ref[r], sp) + _BLK - 1) // _BLK
        blo.append(lo_b)
        cnt.append(hi_b - lo_b)
    cum = [jnp.int32(0)]
    for r in range(_B):
        cum.append(cum[r] + cnt[r])
    T = cum[_B]

    def decode(t):
        row = jnp.int32(0)
        base = jnp.int32(0)
        for r in range(1, _B):
            sel = (t >= cum[r]).astype(jnp.int32)
            row = row + sel
            base = base + sel * cnt[r - 1]
        return row, t - base

    def blo_of(row):
        v = blo[0]
        for r in range(1, _B):
            v = jnp.where(row == r, blo[r], v)
        return v

    def lim_of(row, ref):
        v = ref[0]
        for r in range(1, _B):
            v = jnp.where(row == r, ref[r], v)
        return v

    def issue(t):
        @pl.when(t < T)
        def _():
            row, j = decode(t)
            blk = blo_of(row) + j
            slot = lax.rem(t, _NBUF)
            pltpu.make_async_copy(
                seq.at[row, pl.ds(blk * _BLK, _BLK)],
                buf.at[slot], sems.at[slot]).start()

    out[...] = sc_part[...]
    for i in range(_NBUF):
        issue(jnp.int32(i))

    def body(t, carry):
        slot = lax.rem(t, _NBUF)
        row, j = decode(t)
        blk = blo_of(row) + j
        pltpu.make_async_copy(
            seq.at[row, pl.ds(blk * _BLK, _BLK)],
            buf.at[slot], sems.at[slot]).wait()
        sp = jnp.maximum(lim_of(row, begin_ref), _SPLIT)
        ep = lim_of(row, end_ref)
        lo = jnp.clip(sp - blk * _BLK, 0, _BLK)
        hi = jnp.clip(ep - blk * _BLK, 0, _BLK)
        pos = lax.broadcasted_iota(jnp.int32, (1, _BLK), 1)
        maskf = ((pos >= lo) & (pos < hi)).astype(jnp.float32)
        block = buf[slot]
        ssum = lax.dot_general(
            maskf, block, (((1,), (0,)), ((), ())),
            precision=lax.Precision.DEFAULT)
        out[pl.ds(row, 1), :] += ssum
        issue(t + _NBUF)
        return carry

    lax.fori_loop(0, T, body, jnp.int32(0))

    for r in range(_B):
        lenf = (end_ref[r] - begin_ref[r]).astype(jnp.float32)
        out[r, :] = out[r, :] * (1.0 / lenf)


def _tc_tail(seq, begin, end, sc_part):
    grid_spec = pltpu.PrefetchScalarGridSpec(
        num_scalar_prefetch=2,
        grid=(),
        in_specs=[pl.BlockSpec(memory_space=pl.ANY),
                  pl.BlockSpec(memory_space=pltpu.VMEM)],
        out_specs=pl.BlockSpec(memory_space=pltpu.VMEM),
        scratch_shapes=[
            pltpu.VMEM((_NBUF, _BLK, _D), jnp.float32),
            pltpu.SemaphoreType.DMA((_NBUF,)),
        ],
    )
    return pl.pallas_call(
        _tc_tail_body,
        grid_spec=grid_spec,
        out_shape=jax.ShapeDtypeStruct((_B, _D), jnp.float32),
    )(begin, end, seq, sc_part)


@jax.jit
def kernel(seq, begin, end):
    sc_part = _sc_prefix(seq, begin, end)
    return _tc_tail(seq, begin, end, sc_part)


# speedup vs baseline: 1.2242x; 1.2242x over previous
import jax
import jax.numpy as jnp
from jax import lax
from jax.experimental import pallas as pl
from jax.experimental.pallas import tpu as pltpu
from jax.experimental.pallas import tpu_sc as plsc

_B, _L, _D = 16, 4096, 512
_SPLIT = 256

_CH = 64
_NCH = _L // _CH
_DH = _D // 2
_NJ = _DH // 16
_U = 4

_BLK = 256
_NBUF = 6


def _sc_prefix_body(seq, begin_h, end_h, out, buf0, buf1, bev, env, outb,
                    sem0, sem1):
    c = lax.axis_index("c")
    s = lax.axis_index("s")
    wid = s * 2 + c
    row = wid % _B
    half = wid // _B

    pltpu.sync_copy(begin_h, bev.at[pl.ds(0, 16)])
    pltpu.sync_copy(begin_h, bev.at[pl.ds(16, 16)])
    pltpu.sync_copy(end_h, env.at[pl.ds(0, 16)])
    pltpu.sync_copy(end_h, env.at[pl.ds(16, 16)])
    w_end = jnp.minimum(env[pl.ds(row, 16)][0], _SPLIT)
    w_begin = jnp.minimum(bev[pl.ds(row, 16)][0], w_end)

    c0 = w_begin // _CH
    c1 = (w_end + _CH - 1) // _CH
    bufs = (buf0, buf1)
    sems = (sem0, sem1)

    def issue(ci, b):
        cc = jnp.clip(ci, 0, _NCH - 1)
        pltpu.async_copy(
            seq.at[row, pl.ds(cc * _CH, _CH), pl.ds(half * _DH, _DH)],
            bufs[b], sems[b])

    def wait(b):
        pltpu.make_async_copy(
            seq.at[0, pl.ds(0, _CH), pl.ds(0, _DH)], bufs[b], sems[b]).wait()

    issue(c0, 0)
    issue(c0 + 1, 1)

    def accum_chunk(ci, b, accs):
        buf = bufs[b]
        wait(b)
        lo = jnp.clip(w_begin - ci * _CH, 0, _CH)
        hi = jnp.clip(w_end - ci * _CH, 0, _CH)
        n = hi - lo
        n_main = n - n % _U

        def main_body(k, a):
            p = lo + k * _U
            for u in range(_U):
                a = tuple(a[j] + buf[p + u, pl.ds(j * 16, 16)]
                          for j in range(_NJ))
            return a

        accs = lax.fori_loop(0, n_main // _U, main_body, accs)

        def tail_body(p, a):
            return tuple(a[j] + buf[p, pl.ds(j * 16, 16)] for j in range(_NJ))

        accs = lax.fori_loop(lo + n_main, hi, tail_body, accs)
        issue(ci + 2, b)
        return accs

    def pair_body(k, accs):
        ci = c0 + 2 * k
        accs = accum_chunk(ci, 0, accs)
        accs = accum_chunk(ci + 1, 1, accs)
        return accs

    n_pairs = jnp.maximum(c1 - c0 + 1, 0) // 2
    accs0 = tuple(jnp.zeros((16,), jnp.float32) for _ in range(_NJ))
    accs = lax.fori_loop(0, n_pairs, pair_body, accs0)

    wait(0)
    wait(1)

    for j in range(_NJ):
        outb[pl.ds(j * 16, 16)] = accs[j]
    pltpu.sync_copy(outb, out.at[row, pl.ds(half * _DH, _DH)])


def _sc_prefix(seq, begin, end):
    mesh = plsc.VectorSubcoreMesh(
        core_axis_name="c", subcore_axis_name="s", num_cores=2,
        num_subcores=16)
    run = pl.kernel(
        _sc_prefix_body,
        out_type=jax.ShapeDtypeStruct((_B, _D), jnp.float32),
        mesh=mesh,
        scratch_types=[
            pltpu.VMEM((_CH, _DH), jnp.float32),
            pltpu.VMEM((_CH, _DH), jnp.float32),
            pltpu.VMEM((32,), jnp.int32),
            pltpu.VMEM((32,), jnp.int32),
            pltpu.VMEM((_DH,), jnp.float32),
            pltpu.SemaphoreType.DMA,
            pltpu.SemaphoreType.DMA,
        ],
    )
    return run(seq, begin, end)


def _tc_tail_body(begin_ref, end_ref, seq, out, buf, sems, row_tab, blk_tab):
    cum = jnp.int32(0)
    for r in range(_B):
        sp = jnp.maximum(begin_ref[r], _SPLIT)
        lo_b = sp // _BLK
        hi_b = (jnp.maximum(end_ref[r], sp) + _BLK - 1) // _BLK

        def fill(j, c):
            row_tab[c + j] = jnp.int32(r)
            blk_tab[c + j] = lo_b + j
            return c

        lax.fori_loop(0, hi_b - lo_b, fill, cum)
        cum = cum + (hi_b - lo_b)
    T = cum

    def issue(t):
        @pl.when(t < T)
        def _():
            row = row_tab[t]
            blk = blk_tab[t]
            slot = lax.rem(t, _NBUF)
            pltpu.make_async_copy(
                seq.at[row, pl.ds(blk * _BLK, _BLK)],
                buf.at[slot], sems.at[slot]).start()

    out[...] = jnp.zeros((_B, _D), jnp.float32)
    for i in range(_NBUF):
        issue(jnp.int32(i))

    def body(t, carry):
        slot = lax.rem(t, _NBUF)
        row = row_tab[t]
        blk = blk_tab[t]
        pltpu.make_async_copy(
            seq.at[row, pl.ds(blk * _BLK, _BLK)],
            buf.at[slot], sems.at[slot]).wait()
        sp = jnp.maximum(begin_ref[row], _SPLIT)
        ep = end_ref[row]
        lo = jnp.clip(sp - blk * _BLK, 0, _BLK)
        hi = jnp.clip(ep - blk * _BLK, 0, _BLK)
        pos = lax.broadcasted_iota(jnp.int32, (1, _BLK), 1)
        maskf = ((pos >= lo) & (pos < hi)).astype(jnp.float32)
        block = buf[slot]
        ssum = lax.dot_general(
            maskf, block, (((1,), (0,)), ((), ())),
            precision=lax.Precision.DEFAULT)
        out[pl.ds(row, 1), :] += ssum
        issue(t + _NBUF)
        return carry

    lax.fori_loop(0, T, body, jnp.int32(0))


def _tc_tail(seq, begin, end):
    grid_spec = pltpu.PrefetchScalarGridSpec(
        num_scalar_prefetch=2,
        grid=(),
        in_specs=[pl.BlockSpec(memory_space=pl.ANY)],
        out_specs=pl.BlockSpec(memory_space=pltpu.VMEM),
        scratch_shapes=[
            pltpu.VMEM((_NBUF, _BLK, _D), jnp.float32),
            pltpu.SemaphoreType.DMA((_NBUF,)),
            pltpu.SMEM((_B * (_L // _BLK),), jnp.int32),
            pltpu.SMEM((_B * (_L // _BLK),), jnp.int32),
        ],
    )
    return pl.pallas_call(
        _tc_tail_body,
        grid_spec=grid_spec,
        out_shape=jax.ShapeDtypeStruct((_B, _D), jnp.float32),
    )(begin, end, seq)


def _combine_body(begin_ref, end_ref, a_ref, b_ref, out_ref):
    for r in range(_B):
        lenf = (end_ref[r] - begin_ref[r]).astype(jnp.float32)
        out_ref[r, :] = (a_ref[r, :] + b_ref[r, :]) * (1.0 / lenf)


def _combine(sc_part, tc_part, begin, end):
    grid_spec = pltpu.PrefetchScalarGridSpec(
        num_scalar_prefetch=2,
        grid=(),
        in_specs=[pl.BlockSpec(memory_space=pltpu.VMEM),
                  pl.BlockSpec(memory_space=pltpu.VMEM)],
        out_specs=pl.BlockSpec(memory_space=pltpu.VMEM),
    )
    return pl.pallas_call(
        _combine_body,
        grid_spec=grid_spec,
        out_shape=jax.ShapeDtypeStruct((_B, _D), jnp.float32),
    )(begin, end, sc_part, tc_part)


@jax.jit
def kernel(seq, begin, end):
    sc_part = _sc_prefix(seq, begin, end)
    tc_part = _tc_tail(seq, begin, end)
    return _combine(sc_part, tc_part, begin, end)


# speedup vs baseline: 1.2582x; 1.0278x over previous
import jax
import jax.numpy as jnp
from jax import lax
from jax.experimental import pallas as pl
from jax.experimental.pallas import tpu as pltpu
from jax.experimental.pallas import tpu_sc as plsc

_B, _L, _D = 16, 4096, 512
_SPLIT = 512

_CH = 64
_NCH = _L // _CH
_DH = _D // 2
_NJ = _DH // 16
_U = 4

_BLK = 256
_NBUF = 6


def _sc_prefix_body(seq, begin_h, end_h, out, buf0, buf1, bev, env, outb,
                    sem0, sem1):
    c = lax.axis_index("c")
    s = lax.axis_index("s")
    wid = s * 2 + c
    row = wid % _B
    half = wid // _B

    pltpu.sync_copy(begin_h, bev.at[pl.ds(0, 16)])
    pltpu.sync_copy(begin_h, bev.at[pl.ds(16, 16)])
    pltpu.sync_copy(end_h, env.at[pl.ds(0, 16)])
    pltpu.sync_copy(end_h, env.at[pl.ds(16, 16)])
    w_end = jnp.minimum(env[pl.ds(row, 16)][0], _SPLIT)
    w_begin = jnp.minimum(bev[pl.ds(row, 16)][0], w_end)

    c0 = w_begin // _CH
    c1 = (w_end + _CH - 1) // _CH
    bufs = (buf0, buf1)
    sems = (sem0, sem1)

    def issue(ci, b):
        cc = jnp.clip(ci, 0, _NCH - 1)
        pltpu.async_copy(
            seq.at[row, pl.ds(cc * _CH, _CH), pl.ds(half * _DH, _DH)],
            bufs[b], sems[b])

    def wait(b):
        pltpu.make_async_copy(
            seq.at[0, pl.ds(0, _CH), pl.ds(0, _DH)], bufs[b], sems[b]).wait()

    issue(c0, 0)
    issue(c0 + 1, 1)

    def accum_chunk(ci, b, accs):
        buf = bufs[b]
        wait(b)
        lo = jnp.clip(w_begin - ci * _CH, 0, _CH)
        hi = jnp.clip(w_end - ci * _CH, 0, _CH)
        n = hi - lo
        n_main = n - n % _U

        def main_body(k, a):
            p = lo + k * _U
            for u in range(_U):
                a = tuple(a[j] + buf[p + u, pl.ds(j * 16, 16)]
                          for j in range(_NJ))
            return a

        accs = lax.fori_loop(0, n_main // _U, main_body, accs)

        def tail_body(p, a):
            return tuple(a[j] + buf[p, pl.ds(j * 16, 16)] for j in range(_NJ))

        accs = lax.fori_loop(lo + n_main, hi, tail_body, accs)
        issue(ci + 2, b)
        return accs

    def pair_body(k, accs):
        ci = c0 + 2 * k
        accs = accum_chunk(ci, 0, accs)
        accs = accum_chunk(ci + 1, 1, accs)
        return accs

    n_pairs = jnp.maximum(c1 - c0 + 1, 0) // 2
    accs0 = tuple(jnp.zeros((16,), jnp.float32) for _ in range(_NJ))
    accs = lax.fori_loop(0, n_pairs, pair_body, accs0)

    wait(0)
    wait(1)

    for j in range(_NJ):
        outb[pl.ds(j * 16, 16)] = accs[j]
    pltpu.sync_copy(outb, out.at[row, pl.ds(half * _DH, _DH)])


def _sc_prefix(seq, begin, end):
    mesh = plsc.VectorSubcoreMesh(
        core_axis_name="c", subcore_axis_name="s", num_cores=2,
        num_subcores=16)
    run = pl.kernel(
        _sc_prefix_body,
        out_type=jax.ShapeDtypeStruct((_B, _D), jnp.float32),
        mesh=mesh,
        scratch_types=[
            pltpu.VMEM((_CH, _DH), jnp.float32),
            pltpu.VMEM((_CH, _DH), jnp.float32),
            pltpu.VMEM((32,), jnp.int32),
            pltpu.VMEM((32,), jnp.int32),
            pltpu.VMEM((_DH,), jnp.float32),
            pltpu.SemaphoreType.DMA,
            pltpu.SemaphoreType.DMA,
        ],
    )
    return run(seq, begin, end)


def _tc_tail_body(begin_ref, end_ref, seq, out, buf, sems, row_tab, blk_tab):
    cum = jnp.int32(0)
    for r in range(_B):
        sp = jnp.maximum(begin_ref[r], _SPLIT)
        lo_b = sp // _BLK
        hi_b = (jnp.maximum(end_ref[r], sp) + _BLK - 1) // _BLK

        def fill(j, c):
            row_tab[c + j] = jnp.int32(r)
            blk_tab[c + j] = lo_b + j
            return c

        lax.fori_loop(0, hi_b - lo_b, fill, cum)
        cum = cum + (hi_b - lo_b)
    T = cum

    def issue(t):
        @pl.when(t < T)
        def _():
            row = row_tab[t]
            blk = blk_tab[t]
            slot = lax.rem(t, _NBUF)
            pltpu.make_async_copy(
                seq.at[row, pl.ds(blk * _BLK, _BLK)],
                buf.at[slot], sems.at[slot]).start()

    out[...] = jnp.zeros((_B, _D), jnp.float32)
    for i in range(_NBUF):
        issue(jnp.int32(i))

    def body(t, carry):
        slot = lax.rem(t, _NBUF)
        row = row_tab[t]
        blk = blk_tab[t]
        pltpu.make_async_copy(
            seq.at[row, pl.ds(blk * _BLK, _BLK)],
            buf.at[slot], sems.at[slot]).wait()
        sp = jnp.maximum(begin_ref[row], _SPLIT)
        ep = end_ref[row]
        lo = jnp.clip(sp - blk * _BLK, 0, _BLK)
        hi = jnp.clip(ep - blk * _BLK, 0, _BLK)
        pos = lax.broadcasted_iota(jnp.int32, (1, _BLK), 1)
        maskf = ((pos >= lo) & (pos < hi)).astype(jnp.float32)
        block = buf[slot]
        ssum = lax.dot_general(
            maskf, block, (((1,), (0,)), ((), ())),
            precision=lax.Precision.DEFAULT)
        out[pl.ds(row, 1), :] += ssum
        issue(t + _NBUF)
        return carry

    lax.fori_loop(0, T, body, jnp.int32(0))


def _tc_tail(seq, begin, end):
    grid_spec = pltpu.PrefetchScalarGridSpec(
        num_scalar_prefetch=2,
        grid=(),
        in_specs=[pl.BlockSpec(memory_space=pl.ANY)],
        out_specs=pl.BlockSpec(memory_space=pltpu.VMEM),
        scratch_shapes=[
            pltpu.VMEM((_NBUF, _BLK, _D), jnp.float32),
            pltpu.SemaphoreType.DMA((_NBUF,)),
            pltpu.SMEM((_B * (_L // _BLK),), jnp.int32),
            pltpu.SMEM((_B * (_L // _BLK),), jnp.int32),
        ],
    )
    return pl.pallas_call(
        _tc_tail_body,
        grid_spec=grid_spec,
        out_shape=jax.ShapeDtypeStruct((_B, _D), jnp.float32),
    )(begin, end, seq)


def _combine_body(begin_ref, end_ref, a_ref, b_ref, out_ref):
    for r in range(_B):
        lenf = (end_ref[r] - begin_ref[r]).astype(jnp.float32)
        out_ref[r, :] = (a_ref[r, :] + b_ref[r, :]) * (1.0 / lenf)


def _combine(sc_part, tc_part, begin, end):
    grid_spec = pltpu.PrefetchScalarGridSpec(
        num_scalar_prefetch=2,
        grid=(),
        in_specs=[pl.BlockSpec(memory_space=pltpu.VMEM),
                  pl.BlockSpec(memory_space=pltpu.VMEM)],
        out_specs=pl.BlockSpec(memory_space=pltpu.VMEM),
    )
    return pl.pallas_call(
        _combine_body,
        grid_spec=grid_spec,
        out_shape=jax.ShapeDtypeStruct((_B, _D), jnp.float32),
    )(begin, end, sc_part, tc_part)


@jax.jit
def kernel(seq, begin, end):
    sc_part = _sc_prefix(seq, begin, end)
    tc_part = _tc_tail(seq, begin, end)
    return _combine(sc_part, tc_part, begin, end)
